# baseline (speedup 1.0000x reference)
import functools

import jax
import jax.numpy as jnp
from jax import lax
from jax.experimental import pallas as pl
from jax.experimental.pallas import tpu as pltpu
from jax.experimental.pallas import tpu_sc as plsc

N = 10000
E = 320000
D = 128
NC = 2
NS = 16
NW = NC * NS
EPW = E // NW
BATCH = 128
NBW = 80
PADE = NBW * BATCH - EPW
NT = N + 16
RQ = 624
TAIL_BASE = RQ * NS
TAIL = N - TAIL_BASE

_mesh = plsc.VectorSubcoreMesh(core_axis_name="c", subcore_axis_name="s")


def _striped_copy(src, dst, s):
    base = s * RQ
    pltpu.sync_copy(src.at[pl.ds(base, RQ)], dst.at[pl.ds(base, RQ)])

    @pl.when(s == NS - 1)
    def _():
        pltpu.sync_copy(src.at[pl.ds(TAIL_BASE, TAIL)],
                        dst.at[pl.ds(TAIL_BASE, TAIL)])


def _deg_body(idx_hbm, ones_hbm, zeros_hbm, deg_hbm, shared_deg, idx_v, ones_v):
    c = lax.axis_index("c")
    s = lax.axis_index("s")
    wid = s * NC + c
    _striped_copy(zeros_hbm, shared_deg, s)
    pltpu.sync_copy(ones_hbm, ones_v)
    pltpu.sync_copy(idx_hbm.at[wid], idx_v)
    plsc.subcore_barrier()

    def body(g, carry):
        pltpu.sync_copy(ones_v, shared_deg.at[idx_v.at[g, 1]], add=True)
        return carry

    lax.fori_loop(0, NBW, body, 0)
    plsc.subcore_barrier()
    _striped_copy(shared_deg, deg_hbm.at[c], s)


_deg_kernel = functools.partial(
    pl.kernel,
    out_type=jax.ShapeDtypeStruct((NC, N, D), jnp.float32),
    mesh=_mesh,
    scratch_types=[
        pltpu.VMEM_SHARED((NT, D), jnp.float32),
        pltpu.VMEM((NBW, 2, BATCH), jnp.int32),
        pltpu.VMEM((BATCH, D), jnp.float32),
    ],
)(_deg_body)


NR = NBW // 2


def _agg_body(idx_hbm, y_hbm, zeros_hbm, agg_hbm,
              shared_agg, i0, i1, r0, r1, gsem, is0, is1):
    c = lax.axis_index("c")
    s = lax.axis_index("s")
    wid = s * NC + c

    @pl.when(c == 0)
    def _():
        _striped_copy(y_hbm, shared_agg, s)

    @pl.when(c != 0)
    def _():
        _striped_copy(zeros_hbm, shared_agg, s)

    pltpu.sync_copy(idx_hbm.at[wid, 0], i0)
    pltpu.sync_copy(idx_hbm.at[wid, 1], i1)
    plsc.subcore_barrier()
    pltpu.async_copy(y_hbm.at[i0.at[0]], r0, gsem)

    def body(r, carry):
        g0 = 2 * r
        pltpu.make_async_copy(y_hbm.at[i0.at[0]], r0, gsem).wait()

        @pl.when(r > 0)
        def _():
            pltpu.make_async_copy(idx_hbm.at[wid, g0 + 1], i1, is1).wait()

        pltpu.sync_copy(r0, shared_agg.at[i0.at[1]], add=True)
        pltpu.async_copy(y_hbm.at[i1.at[0]], r1, gsem)

        @pl.when(r < NR - 1)
        def _():
            pltpu.async_copy(idx_hbm.at[wid, g0 + 2], i0, is0)

        pltpu.make_async_copy(y_hbm.at[i1.at[0]], r1, gsem).wait()

        @pl.when(r < NR - 1)
        def _():
            pltpu.make_async_copy(idx_hbm.at[wid, g0 + 2], i0, is0).wait()

        pltpu.sync_copy(r1, shared_agg.at[i1.at[1]], add=True)

        @pl.when(r < NR - 1)
        def _():
            pltpu.async_copy(y_hbm.at[i0.at[0]], r0, gsem)

        @pl.when(r < NR - 1)
        def _():
            pltpu.async_copy(idx_hbm.at[wid, g0 + 3], i1, is1)

        return carry

    lax.fori_loop(0, NR, body, 0)
    plsc.subcore_barrier()
    _striped_copy(shared_agg, agg_hbm.at[c], s)


_agg_kernel = functools.partial(
    pl.kernel,
    out_type=jax.ShapeDtypeStruct((NC, N, D), jnp.float32),
    mesh=_mesh,
    scratch_types=[
        pltpu.VMEM_SHARED((NT, D), jnp.float32),
        pltpu.VMEM((2, BATCH), jnp.int32),
        pltpu.VMEM((2, BATCH), jnp.int32),
        pltpu.VMEM((BATCH, D), jnp.float32),
        pltpu.VMEM((BATCH, D), jnp.float32),
        pltpu.SemaphoreType.DMA,
        pltpu.SemaphoreType.DMA,
        pltpu.SemaphoreType.DMA,
    ],
)(_agg_body)


def _mm_body(x_ref, w_ref, deg_ref, y_ref):
    deg = deg_ref[0, :, 0:1] + deg_ref[1, :, 0:1] + 1.0
    dis = lax.rsqrt(deg)
    xw = jnp.dot(x_ref[...], w_ref[...], preferred_element_type=jnp.float32)
    y_ref[...] = xw * dis


def _mm_kernel(x, w, deg):
    return pl.pallas_call(
        _mm_body,
        out_shape=jax.ShapeDtypeStruct((N, D), jnp.float32),
    )(x, w, deg)


def _fin_body(agg_ref, deg_ref, b_ref, out_ref):
    dis = lax.rsqrt(deg_ref[0, :, 0:1] + deg_ref[1, :, 0:1] + 1.0)
    out_ref[...] = (agg_ref[0] + agg_ref[1]) * dis + b_ref[...]


def _fin_kernel(agg, deg, b):
    return pl.pallas_call(
        _fin_body,
        out_shape=jax.ShapeDtypeStruct((N, D), jnp.float32),
    )(agg, deg, b)


def kernel(x, edge_index, W, b):
    ei = edge_index.astype(jnp.int32)
    src = ei[0].reshape(NW, EPW)
    dst = ei[1].reshape(NW, EPW)
    src = jnp.concatenate(
        [src, jnp.zeros((NW, PADE), jnp.int32)], axis=1).reshape(NW, NBW, BATCH)
    pad_dst = N + jnp.tile(jnp.arange(16, dtype=jnp.int32), PADE // 16)
    dst = jnp.concatenate(
        [dst, jnp.broadcast_to(pad_dst, (NW, PADE))], axis=1).reshape(NW, NBW, BATCH)
    idx = jnp.stack([src, dst], axis=2)
    ones = jnp.ones((BATCH, D), jnp.float32)
    zeros = jnp.zeros((N, D), jnp.float32)

    deg2 = _deg_kernel(idx, ones, zeros)
    y = _mm_kernel(x, W, deg2)
    agg2 = _agg_kernel(idx, y, zeros)
    return _fin_kernel(agg2, deg2, b.reshape(1, D))


# speedup vs baseline: 1.0001x; 1.0001x over previous
import functools

import jax
import jax.numpy as jnp
from jax import lax
from jax.experimental import pallas as pl
from jax.experimental.pallas import tpu as pltpu
from jax.experimental.pallas import tpu_sc as plsc

N = 10000
E = 320000
D = 128
NC = 2
NS = 16
NW = NC * NS
EPW = E // NW
BATCH = 128
NBW = 80
PADE = NBW * BATCH - EPW
NT = N + 16
RQ = 624
TAIL_BASE = RQ * NS
TAIL = N - TAIL_BASE

_mesh = plsc.VectorSubcoreMesh(core_axis_name="c", subcore_axis_name="s")


def _striped_copy(src, dst, s):
    base = s * RQ
    pltpu.sync_copy(src.at[pl.ds(base, RQ)], dst.at[pl.ds(base, RQ)])

    @pl.when(s == NS - 1)
    def _():
        pltpu.sync_copy(src.at[pl.ds(TAIL_BASE, TAIL)],
                        dst.at[pl.ds(TAIL_BASE, TAIL)])


def _deg_body(idx_hbm, ones_hbm, zeros_hbm, deg_hbm, shared_deg, idx_v, ones_v):
    c = lax.axis_index("c")
    s = lax.axis_index("s")
    wid = s * NC + c
    _striped_copy(zeros_hbm, shared_deg, s)
    pltpu.sync_copy(ones_hbm, ones_v)
    pltpu.sync_copy(idx_hbm.at[wid], idx_v)
    plsc.subcore_barrier()

    def body(g, carry):
        pltpu.sync_copy(ones_v, shared_deg.at[idx_v.at[g, 1]], add=True)
        return carry

    lax.fori_loop(0, NBW, body, 0)
    plsc.subcore_barrier()
    _striped_copy(shared_deg, deg_hbm.at[c], s)


_deg_kernel = functools.partial(
    pl.kernel,
    out_type=jax.ShapeDtypeStruct((NC, N, D), jnp.float32),
    mesh=_mesh,
    scratch_types=[
        pltpu.VMEM_SHARED((NT, D), jnp.float32),
        pltpu.VMEM((NBW, 2, BATCH), jnp.int32),
        pltpu.VMEM((BATCH, D), jnp.float32),
    ],
)(_deg_body)


NR = NBW // 2


def _agg_body(idx_hbm, y_hbm, zeros_hbm, agg_hbm,
              shared_agg, idx_v, rows_v, gsem):
    c = lax.axis_index("c")
    s = lax.axis_index("s")
    wid = s * NC + c

    @pl.when(c == 0)
    def _():
        _striped_copy(y_hbm, shared_agg, s)

    @pl.when(c != 0)
    def _():
        _striped_copy(zeros_hbm, shared_agg, s)

    pltpu.sync_copy(idx_hbm.at[wid], idx_v)
    plsc.subcore_barrier()

    def body(g, carry):
        pltpu.async_copy(y_hbm.at[idx_v.at[g, 0]], rows_v, gsem).wait()
        pltpu.sync_copy(rows_v, shared_agg.at[idx_v.at[g, 1]], add=True)
        return carry

    lax.fori_loop(0, NBW, body, 0)
    plsc.subcore_barrier()
    _striped_copy(shared_agg, agg_hbm.at[c], s)


_agg_kernel = functools.partial(
    pl.kernel,
    out_type=jax.ShapeDtypeStruct((NC, N, D), jnp.float32),
    mesh=_mesh,
    scratch_types=[
        pltpu.VMEM_SHARED((NT, D), jnp.float32),
        pltpu.VMEM((NBW, 2, BATCH), jnp.int32),
        pltpu.VMEM((BATCH, D), jnp.float32),
        pltpu.SemaphoreType.DMA,
    ],
)(_agg_body)


def _mm_body(x_ref, w_ref, deg_ref, y_ref):
    deg = deg_ref[0, :, 0:1] + deg_ref[1, :, 0:1] + 1.0
    dis = lax.rsqrt(deg)
    xw = jnp.dot(x_ref[...], w_ref[...], preferred_element_type=jnp.float32)
    y_ref[...] = xw * dis


def _mm_kernel(x, w, deg):
    return pl.pallas_call(
        _mm_body,
        out_shape=jax.ShapeDtypeStruct((N, D), jnp.float32),
    )(x, w, deg)


def _fin_body(agg_ref, deg_ref, b_ref, out_ref):
    dis = lax.rsqrt(deg_ref[0, :, 0:1] + deg_ref[1, :, 0:1] + 1.0)
    out_ref[...] = (agg_ref[0] + agg_ref[1]) * dis + b_ref[...]


def _fin_kernel(agg, deg, b):
    return pl.pallas_call(
        _fin_body,
        out_shape=jax.ShapeDtypeStruct((N, D), jnp.float32),
    )(agg, deg, b)


def kernel(x, edge_index, W, b):
    ei = edge_index.astype(jnp.int32)
    src = ei[0].reshape(NW, EPW)
    dst = ei[1].reshape(NW, EPW)
    src = jnp.concatenate(
        [src, jnp.zeros((NW, PADE), jnp.int32)], axis=1).reshape(NW, NBW, BATCH)
    pad_dst = N + jnp.tile(jnp.arange(16, dtype=jnp.int32), PADE // 16)
    dst = jnp.concatenate(
        [dst, jnp.broadcast_to(pad_dst, (NW, PADE))], axis=1).reshape(NW, NBW, BATCH)
    idx = jnp.stack([src, dst], axis=2)
    ones = jnp.ones((BATCH, D), jnp.float32)
    zeros = jnp.zeros((N, D), jnp.float32)

    deg2 = _deg_kernel(idx, ones, zeros)
    y = _mm_kernel(x, W, deg2)
    agg2 = _agg_kernel(idx, y, zeros)
    return _fin_kernel(agg2, deg2, b.reshape(1, D))


# speedup vs baseline: 1.5909x; 1.5907x over previous
import functools

import jax
import jax.numpy as jnp
from jax import lax
from jax.experimental import pallas as pl
from jax.experimental.pallas import tpu as pltpu
from jax.experimental.pallas import tpu_sc as plsc

N = 10000
E = 320000
D = 128
NC = 2
NS = 16
NW = NC * NS
EPW = E // NW
BATCH = 80
NBW = 126
PADE = NBW * BATCH - EPW
NT = N + 16
RQ = 624
TAIL_BASE = RQ * NS
TAIL = N - TAIL_BASE

_mesh = plsc.VectorSubcoreMesh(core_axis_name="c", subcore_axis_name="s")


def _striped_copy(src, dst, s):
    base = s * RQ
    pltpu.sync_copy(src.at[pl.ds(base, RQ)], dst.at[pl.ds(base, RQ)])

    @pl.when(s == NS - 1)
    def _():
        pltpu.sync_copy(src.at[pl.ds(TAIL_BASE, TAIL)],
                        dst.at[pl.ds(TAIL_BASE, TAIL)])


def _deg_body(didx_hbm, ones_hbm, zeros_hbm, deg_hbm, shared_deg, idx_v, ones_v):
    c = lax.axis_index("c")
    s = lax.axis_index("s")
    wid = s * NC + c
    _striped_copy(zeros_hbm, shared_deg, s)
    pltpu.sync_copy(ones_hbm, ones_v)
    pltpu.sync_copy(didx_hbm.at[wid], idx_v)
    plsc.subcore_barrier()

    def body(g, carry):
        pltpu.sync_copy(ones_v, shared_deg.at[idx_v.at[g]], add=True)
        return carry

    lax.fori_loop(0, NBW, body, 0)
    plsc.subcore_barrier()
    _striped_copy(shared_deg, deg_hbm.at[c], s)


_deg_kernel = functools.partial(
    pl.kernel,
    out_type=jax.ShapeDtypeStruct((NC, N, D), jnp.float32),
    mesh=_mesh,
    scratch_types=[
        pltpu.VMEM_SHARED((NT, D), jnp.float32),
        pltpu.VMEM((NBW, BATCH), jnp.int32),
        pltpu.VMEM((BATCH, D), jnp.float32),
    ],
)(_deg_body)


NR = NBW // 2


NR = NBW // 2


def _agg_body(sidx_hbm, didx_hbm, y_hbm, zeros_hbm, agg_hbm,
              shared_agg, sidx_v, d_v, r0, r1, gsem, dsem0, dsem1):
    c = lax.axis_index("c")
    s = lax.axis_index("s")
    wid = s * NC + c

    @pl.when(c == 0)
    def _():
        _striped_copy(y_hbm, shared_agg, s)

    @pl.when(c != 0)
    def _():
        _striped_copy(zeros_hbm, shared_agg, s)

    pltpu.sync_copy(sidx_hbm.at[wid], sidx_v)
    pltpu.sync_copy(didx_hbm.at[wid, 0], d_v.at[0])
    pltpu.sync_copy(didx_hbm.at[wid, 1], d_v.at[1])
    plsc.subcore_barrier()
    pltpu.async_copy(y_hbm.at[sidx_v.at[0]], r0, gsem)

    def body(r, carry):
        g0 = 2 * r
        pltpu.make_async_copy(y_hbm.at[sidx_v.at[g0]], r0, gsem).wait()
        pltpu.async_copy(y_hbm.at[sidx_v.at[g0 + 1]], r1, gsem)

        @pl.when(r > 0)
        def _():
            pltpu.make_async_copy(didx_hbm.at[wid, g0], d_v.at[0], dsem0).wait()

        pltpu.sync_copy(r0, shared_agg.at[d_v.at[0]], add=True)

        @pl.when(r < NR - 1)
        def _():
            pltpu.async_copy(didx_hbm.at[wid, g0 + 2], d_v.at[0], dsem0)

        pltpu.make_async_copy(y_hbm.at[sidx_v.at[g0 + 1]], r1, gsem).wait()

        @pl.when(r < NR - 1)
        def _():
            pltpu.async_copy(y_hbm.at[sidx_v.at[g0 + 2]], r0, gsem)

        @pl.when(r > 0)
        def _():
            pltpu.make_async_copy(didx_hbm.at[wid, g0 + 1], d_v.at[1], dsem1).wait()

        pltpu.sync_copy(r1, shared_agg.at[d_v.at[1]], add=True)

        @pl.when(r < NR - 1)
        def _():
            pltpu.async_copy(didx_hbm.at[wid, g0 + 3], d_v.at[1], dsem1)

        return carry

    lax.fori_loop(0, NR, body, 0)
    plsc.subcore_barrier()
    _striped_copy(shared_agg, agg_hbm.at[c], s)


_agg_kernel = functools.partial(
    pl.kernel,
    out_type=jax.ShapeDtypeStruct((NC, N, D), jnp.float32),
    mesh=_mesh,
    scratch_types=[
        pltpu.VMEM_SHARED((NT, D), jnp.float32),
        pltpu.VMEM((NBW, BATCH), jnp.int32),
        pltpu.VMEM((2, BATCH), jnp.int32),
        pltpu.VMEM((BATCH, D), jnp.float32),
        pltpu.VMEM((BATCH, D), jnp.float32),
        pltpu.SemaphoreType.DMA,
        pltpu.SemaphoreType.DMA,
        pltpu.SemaphoreType.DMA,
    ],
)(_agg_body)


def _mm_body(x_ref, w_ref, deg_ref, y_ref):
    deg = deg_ref[0, :, 0:1] + deg_ref[1, :, 0:1] + 1.0
    dis = lax.rsqrt(deg)
    xw = jnp.dot(x_ref[...], w_ref[...], preferred_element_type=jnp.float32)
    y_ref[...] = xw * dis


def _mm_kernel(x, w, deg):
    return pl.pallas_call(
        _mm_body,
        out_shape=jax.ShapeDtypeStruct((N, D), jnp.float32),
    )(x, w, deg)


def _fin_body(agg_ref, deg_ref, b_ref, out_ref):
    dis = lax.rsqrt(deg_ref[0, :, 0:1] + deg_ref[1, :, 0:1] + 1.0)
    out_ref[...] = (agg_ref[0] + agg_ref[1]) * dis + b_ref[...]


def _fin_kernel(agg, deg, b):
    return pl.pallas_call(
        _fin_body,
        out_shape=jax.ShapeDtypeStruct((N, D), jnp.float32),
    )(agg, deg, b)


def kernel(x, edge_index, W, b):
    ei = edge_index.astype(jnp.int32)
    src = ei[0].reshape(NW, EPW)
    dst = ei[1].reshape(NW, EPW)
    src = jnp.concatenate(
        [src, jnp.zeros((NW, PADE), jnp.int32)], axis=1).reshape(NW, NBW, BATCH)
    pad_dst = N + jnp.tile(jnp.arange(16, dtype=jnp.int32), PADE // 16)
    dst = jnp.concatenate(
        [dst, jnp.broadcast_to(pad_dst, (NW, PADE))], axis=1).reshape(NW, NBW, BATCH)
    ones = jnp.ones((BATCH, D), jnp.float32)
    zeros = jnp.zeros((N, D), jnp.float32)

    deg2 = _deg_kernel(dst, ones, zeros)
    y = _mm_kernel(x, W, deg2)
    agg2 = _agg_kernel(src, dst, y, zeros)
    return _fin_kernel(agg2, deg2, b.reshape(1, D))


# speedup vs baseline: 1.5916x; 1.0005x over previous
import functools

import jax
import jax.numpy as jnp
from jax import lax
from jax.experimental import pallas as pl
from jax.experimental.pallas import tpu as pltpu
from jax.experimental.pallas import tpu_sc as plsc

N = 10000
E = 320000
D = 128
NC = 2
NS = 16
NW = NC * NS
EPW = E // NW
BATCH = 80
NBW = 126
PADE = NBW * BATCH - EPW
NT = N + 16
RQ = 624
TAIL_BASE = RQ * NS
TAIL = N - TAIL_BASE

_mesh = plsc.VectorSubcoreMesh(core_axis_name="c", subcore_axis_name="s")


def _striped_copy(src, dst, s):
    base = s * RQ
    pltpu.sync_copy(src.at[pl.ds(base, RQ)], dst.at[pl.ds(base, RQ)])

    @pl.when(s == NS - 1)
    def _():
        pltpu.sync_copy(src.at[pl.ds(TAIL_BASE, TAIL)],
                        dst.at[pl.ds(TAIL_BASE, TAIL)])


def _deg_body(didx_hbm, ones_hbm, zeros_hbm, deg_hbm, shared_deg, idx_v, ones_v):
    c = lax.axis_index("c")
    s = lax.axis_index("s")
    wid = s * NC + c
    _striped_copy(zeros_hbm, shared_deg, s)
    pltpu.sync_copy(ones_hbm, ones_v)
    pltpu.sync_copy(didx_hbm.at[wid], idx_v)
    plsc.subcore_barrier()

    def body(g, carry):
        pltpu.sync_copy(ones_v, shared_deg.at[idx_v.at[g]], add=True)
        return carry

    lax.fori_loop(0, NBW, body, 0)
    plsc.subcore_barrier()
    _striped_copy(shared_deg, deg_hbm.at[c], s)


_deg_kernel = functools.partial(
    pl.kernel,
    out_type=jax.ShapeDtypeStruct((NC, N, D), jnp.float32),
    mesh=_mesh,
    scratch_types=[
        pltpu.VMEM_SHARED((NT, D), jnp.float32),
        pltpu.VMEM((NBW, BATCH), jnp.int32),
        pltpu.VMEM((BATCH, D), jnp.float32),
    ],
)(_deg_body)


NR = NBW // 2


NR = NBW // 2


def _agg_body(sidx_hbm, didx_hbm, y_hbm, zeros_hbm, agg_hbm,
              shared_agg, sidx_v, d_v, r0, r1,
              gsem0, gsem1, ssem0, ssem1, dsem0, dsem1):
    c = lax.axis_index("c")
    s = lax.axis_index("s")
    wid = s * NC + c

    @pl.when(c == 0)
    def _():
        _striped_copy(y_hbm, shared_agg, s)

    @pl.when(c != 0)
    def _():
        _striped_copy(zeros_hbm, shared_agg, s)

    pltpu.sync_copy(sidx_hbm.at[wid], sidx_v)
    pltpu.sync_copy(didx_hbm.at[wid, 0], d_v.at[0])
    plsc.subcore_barrier()
    pltpu.async_copy(y_hbm.at[sidx_v.at[0]], r0, gsem0)

    def body(r, carry):
        a = 2 * r

        pltpu.make_async_copy(y_hbm.at[sidx_v.at[a]], r0, gsem0).wait()

        @pl.when(r > 0)
        def _():
            pltpu.make_async_copy(didx_hbm.at[wid, a], d_v.at[0], dsem0).wait()

        pltpu.async_copy(r0, shared_agg.at[d_v.at[0]], ssem0, add=True)

        @pl.when(r > 0)
        def _():
            pltpu.make_async_copy(r1, shared_agg.at[d_v.at[1]], ssem1).wait()

        pltpu.async_copy(didx_hbm.at[wid, a + 1], d_v.at[1], dsem1)
        pltpu.async_copy(y_hbm.at[sidx_v.at[a + 1]], r1, gsem1)
        pltpu.make_async_copy(y_hbm.at[sidx_v.at[a + 1]], r1, gsem1).wait()
        pltpu.make_async_copy(didx_hbm.at[wid, a + 1], d_v.at[1], dsem1).wait()
        pltpu.async_copy(r1, shared_agg.at[d_v.at[1]], ssem1, add=True)
        pltpu.make_async_copy(r0, shared_agg.at[d_v.at[0]], ssem0).wait()

        @pl.when(r < NR - 1)
        def _():
            pltpu.async_copy(didx_hbm.at[wid, a + 2], d_v.at[0], dsem0)
            pltpu.async_copy(y_hbm.at[sidx_v.at[a + 2]], r0, gsem0)

        return carry

    lax.fori_loop(0, NR, body, 0)
    pltpu.make_async_copy(r1, shared_agg.at[d_v.at[1]], ssem1).wait()
    plsc.subcore_barrier()
    _striped_copy(shared_agg, agg_hbm.at[c], s)


_agg_kernel = functools.partial(
    pl.kernel,
    out_type=jax.ShapeDtypeStruct((NC, N, D), jnp.float32),
    mesh=_mesh,
    scratch_types=[
        pltpu.VMEM_SHARED((NT, D), jnp.float32),
        pltpu.VMEM((NBW, BATCH), jnp.int32),
        pltpu.VMEM((2, BATCH), jnp.int32),
        pltpu.VMEM((BATCH, D), jnp.float32),
        pltpu.VMEM((BATCH, D), jnp.float32),
        pltpu.SemaphoreType.DMA,
        pltpu.SemaphoreType.DMA,
        pltpu.SemaphoreType.DMA,
        pltpu.SemaphoreType.DMA,
        pltpu.SemaphoreType.DMA,
        pltpu.SemaphoreType.DMA,
    ],
)(_agg_body)


def _mm_body(x_ref, w_ref, deg_ref, y_ref):
    deg = deg_ref[0, :, 0:1] + deg_ref[1, :, 0:1] + 1.0
    dis = lax.rsqrt(deg)
    xw = jnp.dot(x_ref[...], w_ref[...], preferred_element_type=jnp.float32)
    y_ref[...] = xw * dis


def _mm_kernel(x, w, deg):
    return pl.pallas_call(
        _mm_body,
        out_shape=jax.ShapeDtypeStruct((N, D), jnp.float32),
    )(x, w, deg)


def _fin_body(agg_ref, deg_ref, b_ref, out_ref):
    dis = lax.rsqrt(deg_ref[0, :, 0:1] + deg_ref[1, :, 0:1] + 1.0)
    out_ref[...] = (agg_ref[0] + agg_ref[1]) * dis + b_ref[...]


def _fin_kernel(agg, deg, b):
    return pl.pallas_call(
        _fin_body,
        out_shape=jax.ShapeDtypeStruct((N, D), jnp.float32),
    )(agg, deg, b)


def kernel(x, edge_index, W, b):
    ei = edge_index.astype(jnp.int32)
    src = ei[0].reshape(NW, EPW)
    dst = ei[1].reshape(NW, EPW)
    src = jnp.concatenate(
        [src, jnp.zeros((NW, PADE), jnp.int32)], axis=1).reshape(NW, NBW, BATCH)
    pad_dst = N + jnp.tile(jnp.arange(16, dtype=jnp.int32), PADE // 16)
    dst = jnp.concatenate(
        [dst, jnp.broadcast_to(pad_dst, (NW, PADE))], axis=1).reshape(NW, NBW, BATCH)
    ones = jnp.ones((BATCH, D), jnp.float32)
    zeros = jnp.zeros((N, D), jnp.float32)

    deg2 = _deg_kernel(dst, ones, zeros)
    y = _mm_kernel(x, W, deg2)
    agg2 = _agg_kernel(src, dst, y, zeros)
    return _fin_kernel(agg2, deg2, b.reshape(1, D))


# speedup vs baseline: 1.7766x; 1.1163x over previous
import functools

import jax
import jax.numpy as jnp
from jax import lax
from jax.experimental import pallas as pl
from jax.experimental.pallas import tpu as pltpu
from jax.experimental.pallas import tpu_sc as plsc

N = 10000
E = 320000
D = 128
NC = 2
NS = 16
NW = NC * NS
EPW = E // NW
BATCH = 80
NB = EPW // BATCH
DW = 8
RQ = 624
TAIL_BASE = RQ * NS
TAIL = N - TAIL_BASE

_mesh = plsc.VectorSubcoreMesh(core_axis_name="c", subcore_axis_name="s")


def _striped_copy(src, dst, s):
    base = s * RQ
    pltpu.sync_copy(src.at[pl.ds(base, RQ)], dst.at[pl.ds(base, RQ)])

    @pl.when(s == NS - 1)
    def _():
        pltpu.sync_copy(src.at[pl.ds(TAIL_BASE, TAIL)],
                        dst.at[pl.ds(TAIL_BASE, TAIL)])


def _deg_body(didx_hbm, ones_hbm, zeros_hbm, deg_hbm, shared_deg, idx_v, ones_v):
    c = lax.axis_index("c")
    s = lax.axis_index("s")
    wid = s * NC + c
    _striped_copy(zeros_hbm, shared_deg, s)
    pltpu.sync_copy(ones_hbm, ones_v)
    pltpu.sync_copy(didx_hbm.at[wid], idx_v)
    plsc.subcore_barrier()

    def body(g, carry):
        pltpu.sync_copy(ones_v, shared_deg.at[idx_v.at[g]], add=True)
        return carry

    lax.fori_loop(0, NB, body, 0)
    plsc.subcore_barrier()
    _striped_copy(shared_deg, deg_hbm.at[c], s)


_deg_kernel = functools.partial(
    pl.kernel,
    out_type=jax.ShapeDtypeStruct((NC, N, D), jnp.float32),
    mesh=_mesh,
    scratch_types=[
        pltpu.VMEM_SHARED((N, D), jnp.float32),
        pltpu.VMEM((NB, BATCH), jnp.int32),
        pltpu.VMEM((BATCH, D), jnp.float32),
    ],
)(_deg_body)


def _agg_body(sidx_hbm, didx_hbm, y_hbm, zeros_hbm, agg_hbm,
              shared_agg, sidx_v, didx_v, rows_v, gsem):
    c = lax.axis_index("c")
    s = lax.axis_index("s")
    wid = s * NC + c

    @pl.when(c == 0)
    def _():
        _striped_copy(y_hbm, shared_agg, s)

    @pl.when(c != 0)
    def _():
        _striped_copy(zeros_hbm, shared_agg, s)

    pltpu.sync_copy(sidx_hbm.at[wid], sidx_v)
    pltpu.sync_copy(didx_hbm.at[wid], didx_v)
    plsc.subcore_barrier()

    def body(g, carry):
        pltpu.async_copy(y_hbm.at[sidx_v.at[g]], rows_v, gsem).wait()
        pltpu.sync_copy(rows_v, shared_agg.at[didx_v.at[g]], add=True)
        return carry

    lax.fori_loop(0, NB, body, 0)
    plsc.subcore_barrier()
    _striped_copy(shared_agg, agg_hbm.at[c], s)


_agg_kernel = functools.partial(
    pl.kernel,
    out_type=jax.ShapeDtypeStruct((NC, N, D), jnp.float32),
    mesh=_mesh,
    scratch_types=[
        pltpu.VMEM_SHARED((N, D), jnp.float32),
        pltpu.VMEM((NB, BATCH), jnp.int32),
        pltpu.VMEM((NB, BATCH), jnp.int32),
        pltpu.VMEM((BATCH, D), jnp.float32),
        pltpu.SemaphoreType.DMA,
    ],
)(_agg_body)


def _mm_body(x_ref, w_ref, deg_ref, y_ref):
    deg = deg_ref[0, :, 0:1] + deg_ref[1, :, 0:1] + 1.0
    dis = lax.rsqrt(deg)
    xw = jnp.dot(x_ref[...], w_ref[...], preferred_element_type=jnp.float32)
    y_ref[...] = xw * dis


def _mm_kernel(x, w, deg):
    return pl.pallas_call(
        _mm_body,
        out_shape=jax.ShapeDtypeStruct((N, D), jnp.float32),
    )(x, w, deg)


def _fin_body(agg_ref, deg_ref, b_ref, out_ref):
    dis = lax.rsqrt(deg_ref[0, :, 0:1] + deg_ref[1, :, 0:1] + 1.0)
    out_ref[...] = (agg_ref[0] + agg_ref[1]) * dis + b_ref[...]


def _fin_kernel(agg, deg, b):
    return pl.pallas_call(
        _fin_body,
        out_shape=jax.ShapeDtypeStruct((N, D), jnp.float32),
    )(agg, deg, b)


def kernel(x, edge_index, W, b):
    ei = edge_index.astype(jnp.int32)
    src = ei[0].reshape(NW, NB, BATCH)
    dst = ei[1].reshape(NW, NB, BATCH)
    ones = jnp.ones((BATCH, D), jnp.float32)
    zeros = jnp.zeros((N, D), jnp.float32)

    deg2 = _deg_kernel(dst, ones, zeros)
    deg8 = deg2[:, :, :DW]
    y = _mm_kernel(x, W, deg8)
    agg2 = _agg_kernel(src, dst, y, zeros)
    return _fin_kernel(agg2, deg8, b.reshape(1, D))
